# baseline (speedup 1.0000x reference)
import jax
import jax.numpy as jnp
from jax import lax
from jax.experimental import pallas as pl
from jax.experimental.pallas import tpu as pltpu

_VOCAB = 1000000
_OUT_DIM = 64
_BLK = 128


def _tc_body(x_smem, w_ref, o_ref):
    col = x_smem[0] % _BLK
    lane = lax.broadcasted_iota(jnp.int32, (_OUT_DIM, _BLK), 1)
    sel = jnp.where(lane == col, w_ref[...], 0.0)
    o_ref[...] = jnp.sum(sel, axis=1)


_grid_spec = pltpu.PrefetchScalarGridSpec(
    num_scalar_prefetch=1,
    grid=(1,),
    in_specs=[
        pl.BlockSpec((_OUT_DIM, _BLK), lambda i, xs: (0, xs[0] // _BLK)),
    ],
    out_specs=pl.BlockSpec((_OUT_DIM,), lambda i, xs: (0,)),
)

_lookup = pl.pallas_call(
    _tc_body,
    grid_spec=_grid_spec,
    out_shape=jax.ShapeDtypeStruct((_OUT_DIM,), jnp.float32),
)


def kernel(x, W):
    xi = x.astype(jnp.int32).reshape((1,))
    return _lookup(xi, W)
